# baseline (speedup 1.0000x reference)
import functools

import jax
import jax.numpy as jnp
from jax import lax
from jax.experimental import pallas as pl
from jax.experimental.pallas import tpu as pltpu
from jax.experimental.pallas import tpu_sc as plsc

NC, NS = 2, 16
NW = NC * NS
SC_CHUNK = 64
TC_ROWS = 20480
TC_BLK = 4096


def _copy_block(x_ref, o_ref):
    o_ref[...] = x_ref[...]


def kernel(x, W1, b1, W2, b2):
    B, S, D = x.shape
    N = B * S
    xf = x.reshape(N, D)
    sc_rows = N - TC_ROWS
    rows_w = sc_rows // NW
    nchunks = rows_w // SC_CHUNK

    head = xf[:TC_ROWS]
    tail = xf[TC_ROWS:]

    out1 = pl.pallas_call(
        _copy_block,
        grid=(TC_ROWS // TC_BLK,),
        in_specs=[pl.BlockSpec((TC_BLK, D), lambda i: (i, 0))],
        out_specs=pl.BlockSpec((TC_BLK, D), lambda i: (i, 0)),
        out_shape=jax.ShapeDtypeStruct((TC_ROWS, D), x.dtype),
        compiler_params=pltpu.CompilerParams(
            dimension_semantics=("parallel",),
        ),
    )(head)

    mesh = plsc.VectorSubcoreMesh(core_axis_name="c", subcore_axis_name="s")

    @functools.partial(
        pl.kernel,
        out_type=jax.ShapeDtypeStruct((sc_rows, D), jnp.float32),
        mesh=mesh,
        scratch_types=[
            pltpu.VMEM((SC_CHUNK, D), jnp.float32),
            pltpu.VMEM((SC_CHUNK, D), jnp.float32),
            pltpu.SemaphoreType.DMA,
            pltpu.SemaphoreType.DMA,
            pltpu.SemaphoreType.DMA,
            pltpu.SemaphoreType.DMA,
        ],
    )
    def sc_copy(x_hbm, out_hbm, buf0, buf1, si0, si1, so0, so1):
        wid = lax.axis_index("s") * NC + lax.axis_index("c")
        base = wid * rows_w
        bufs = (buf0, buf1)
        sis = (si0, si1)
        sos = (so0, so1)

        def sl(i):
            return pl.ds(base + i * SC_CHUNK, SC_CHUNK)

        ci = [None, None]
        co = [None, None]
        ci[0] = pltpu.async_copy(x_hbm.at[sl(0)], bufs[0], sis[0])
        for i in range(nchunks):
            b = i % 2
            nb = (i + 1) % 2
            if i + 1 < nchunks:
                if co[nb] is not None:
                    co[nb].wait()
                ci[nb] = pltpu.async_copy(x_hbm.at[sl(i + 1)], bufs[nb], sis[nb])
            ci[b].wait()
            co[b] = pltpu.async_copy(bufs[b], out_hbm.at[sl(i)], sos[b])
        co[(nchunks - 1) % 2].wait()
        if nchunks > 1:
            co[nchunks % 2].wait()

    out2 = sc_copy(tail)
    out = jnp.concatenate([out1, out2], axis=0)
    return out.reshape(B, S, D)


# speedup vs baseline: 3.3095x; 3.3095x over previous
import jax
import jax.numpy as jnp
from jax.experimental import pallas as pl
from jax.experimental.pallas import tpu as pltpu


def _copy_block(x_ref, o_ref):
    o_ref[...] = x_ref[...]


def kernel(x, W1, b1, W2, b2):
    B, S, D = x.shape
    N = B * S
    xf = x.reshape(N, D)
    ROWS = 4096
    out = pl.pallas_call(
        _copy_block,
        grid=(N // ROWS,),
        in_specs=[pl.BlockSpec((ROWS, D), lambda i: (i, 0))],
        out_specs=pl.BlockSpec((ROWS, D), lambda i: (i, 0)),
        out_shape=jax.ShapeDtypeStruct((N, D), x.dtype),
        compiler_params=pltpu.CompilerParams(
            dimension_semantics=("parallel",),
        ),
    )(xf)
    return out.reshape(B, S, D)
